# baseline (speedup 1.0000x reference)
import jax
import jax.numpy as jnp
from jax import lax
from jax.experimental import pallas as pl
from jax.experimental.pallas import tpu as pltpu
from jax.experimental.pallas import tpu_sc as plsc

N = 10000
E = 320000
D = 128
D_H = 256

NC = 2
NS = 16
NW = NC * NS
L = 16

CHUNK = 128
NCH = 80
EPT = NCH * CHUNK
E_PAD = NW * EPT
NP = 10240
RPT = NP // NS
NSLAB = 2
HCH = NCH // NSLAB


def _deg_body(dstH, cntH, dst_v, cnt_v):
    cid = lax.axis_index("c")
    sid = lax.axis_index("s")
    w = cid * NS + sid
    pltpu.sync_copy(dstH.at[w], dst_v)

    zeros16 = jnp.zeros((L,), jnp.float32)

    @pl.loop(0, NP // L)
    def _zero(i):
        cnt_v[pl.ds(i * L, L)] = zeros16

    ones16 = jnp.ones((L,), jnp.float32)

    for p in range(NSLAB):
        @pl.loop(0, NCH // NSLAB)
        def _count(j):
            for k in range(CHUNK // L):
                idx = dst_v[p, j, pl.ds(k * L, L)]
                plsc.addupdate_scatter(cnt_v, [idx], ones16)

    pltpu.sync_copy(cnt_v, cntH.at[w])


_deg = pl.kernel(
    _deg_body,
    out_type=jax.ShapeDtypeStruct((NW, NP), jnp.float32),
    mesh=plsc.VectorSubcoreMesh(core_axis_name="c", subcore_axis_name="s"),
    scratch_types=[
        pltpu.VMEM((NSLAB, HCH, CHUNK), jnp.int32),
        pltpu.VMEM((NP,), jnp.float32),
    ],
    compiler_params=pltpu.CompilerParams(needs_layout_passes=False),
)


def _agg_body(yH, srcH, dstH, outH, src_v, dst_v, rows0, rows1, S_sh,
              semg0, semg1, sems0, sems1):
    cid = lax.axis_index("c")
    sid = lax.axis_index("s")
    w = cid * NS + sid

    for p in range(NSLAB):
        pltpu.sync_copy(srcH.at[w].at[p], src_v)
        pltpu.sync_copy(dstH.at[w].at[p], dst_v)
        pltpu.async_copy(yH.at[src_v.at[0]], rows0, semg0)

        if p == 0:
            zeros16 = jnp.zeros((L,), jnp.float32)

            @pl.loop(0, CHUNK)
            def _zero(r):
                for k in range(D // L):
                    rows1[r, pl.ds(k * L, L)] = zeros16

            for t in range(RPT // CHUNK):
                pltpu.async_copy(
                    rows1, S_sh.at[pl.ds(sid * RPT + t * CHUNK, CHUNK)],
                    sems1)
            for t in range(RPT // CHUNK):
                pltpu.make_async_copy(
                    rows1, S_sh.at[pl.ds(sid * RPT + t * CHUNK, CHUNK)],
                    sems1).wait()
            plsc.subcore_barrier()

        pltpu.async_copy(yH.at[src_v.at[1]], rows1, semg1)

        @pl.loop(0, HCH, step=2)
        def _edges(j):
            pltpu.make_async_copy(yH.at[src_v.at[j]], rows0, semg0).wait()
            pltpu.async_copy(rows0, S_sh.at[dst_v.at[j]], sems0, add=True)
            pltpu.make_async_copy(yH.at[src_v.at[j + 1]], rows1, semg1).wait()
            pltpu.async_copy(rows1, S_sh.at[dst_v.at[j + 1]], sems1, add=True)
            pltpu.make_async_copy(rows0, S_sh.at[dst_v.at[j]], sems0).wait()

            @pl.when(j + 2 < HCH)
            def _():
                pltpu.async_copy(yH.at[src_v.at[j + 2]], rows0, semg0)

            pltpu.make_async_copy(rows1, S_sh.at[dst_v.at[j + 1]],
                                  sems1).wait()

            @pl.when(j + 3 < HCH)
            def _():
                pltpu.async_copy(yH.at[src_v.at[j + 3]], rows1, semg1)

    plsc.subcore_barrier()
    pltpu.sync_copy(S_sh.at[pl.ds(sid * RPT, RPT)],
                    outH.at[cid].at[pl.ds(sid * RPT, RPT)])


_agg = pl.kernel(
    _agg_body,
    out_type=jax.ShapeDtypeStruct((NC, NP, D), jnp.float32),
    mesh=plsc.VectorSubcoreMesh(core_axis_name="c", subcore_axis_name="s"),
    scratch_types=[
        pltpu.VMEM((HCH, CHUNK), jnp.int32),
        pltpu.VMEM((HCH, CHUNK), jnp.int32),
        pltpu.VMEM((CHUNK, D), jnp.float32),
        pltpu.VMEM((CHUNK, D), jnp.float32),
        pltpu.VMEM_SHARED((NP, D), jnp.float32),
        pltpu.SemaphoreType.DMA,
        pltpu.SemaphoreType.DMA,
        pltpu.SemaphoreType.DMA,
        pltpu.SemaphoreType.DMA,
    ],
)


PBLK = 512


def _prep_body(cnt_ref, x_ref, y_ref, dinv_ref):
    deg = jnp.sum(cnt_ref[...], axis=0) + 1.0
    dinv = lax.rsqrt(deg)[:, None]
    dinv_ref[...] = dinv
    y_ref[...] = x_ref[...] * dinv


_prep = pl.pallas_call(
    _prep_body,
    grid=(NP // PBLK,),
    in_specs=[
        pl.BlockSpec((NW, PBLK), lambda i: (0, i)),
        pl.BlockSpec((PBLK, D), lambda i: (i, 0)),
    ],
    out_specs=[
        pl.BlockSpec((PBLK, D), lambda i: (i, 0)),
        pl.BlockSpec((PBLK, 1), lambda i: (i, 0)),
    ],
    out_shape=[
        jax.ShapeDtypeStruct((NP, D), jnp.float32),
        jax.ShapeDtypeStruct((NP, 1), jnp.float32),
    ],
)


def _elu(v):
    return jnp.where(v > 0, v, jnp.exp(jnp.minimum(v, 0.0)) - 1.0)


MBLK = 512


def _mid_body(s0, s1, y0, dinv, w1, b1, w2, yh):
    a = (s0[...] + s1[...] + y0[...]) * dinv[...]
    h = _elu(jnp.dot(a, w1[...], preferred_element_type=jnp.float32)
             + b1[...])
    yh[...] = jnp.dot(h, w2[...],
                      preferred_element_type=jnp.float32) * dinv[...]


_mid = pl.pallas_call(
    _mid_body,
    grid=(NP // MBLK,),
    in_specs=[
        pl.BlockSpec((MBLK, D), lambda i: (i, 0)),
        pl.BlockSpec((MBLK, D), lambda i: (i, 0)),
        pl.BlockSpec((MBLK, D), lambda i: (i, 0)),
        pl.BlockSpec((MBLK, 1), lambda i: (i, 0)),
        pl.BlockSpec((D, D_H), lambda i: (0, 0)),
        pl.BlockSpec((1, D_H), lambda i: (0, 0)),
        pl.BlockSpec((D_H, D), lambda i: (0, 0)),
    ],
    out_specs=pl.BlockSpec((MBLK, D), lambda i: (i, 0)),
    out_shape=jax.ShapeDtypeStruct((NP, D), jnp.float32),
)


FBLK = 1000


def _fin_body(t0, t1, yh, dinv, b2, out):
    v = (t0[...] + t1[...] + yh[...]) * dinv[...] + b2[...]
    out[...] = _elu(v)


_fin = pl.pallas_call(
    _fin_body,
    grid=(N // FBLK,),
    in_specs=[
        pl.BlockSpec((FBLK, D), lambda i: (i, 0)),
        pl.BlockSpec((FBLK, D), lambda i: (i, 0)),
        pl.BlockSpec((FBLK, D), lambda i: (i, 0)),
        pl.BlockSpec((FBLK, 1), lambda i: (i, 0)),
        pl.BlockSpec((1, D), lambda i: (0, 0)),
    ],
    out_specs=pl.BlockSpec((FBLK, D), lambda i: (i, 0)),
    out_shape=jax.ShapeDtypeStruct((N, D), jnp.float32),
)


def kernel(x, edge_index, W1, b1, W2, b2):
    src = edge_index[0]
    dst = edge_index[1]
    pad = E_PAD - E
    pad_idx = (N + (jnp.arange(pad, dtype=jnp.int32) % (NP - N)))
    srcp = jnp.concatenate([src, pad_idx]).reshape(NW, NSLAB, HCH, CHUNK)
    dstp = jnp.concatenate([dst, pad_idx]).reshape(NW, NSLAB, HCH, CHUNK)
    x_pad = jnp.pad(x, ((0, NP - N), (0, 0)))

    cnt = _deg(dstp)
    y0, dinv = _prep(cnt, x_pad)
    s = _agg(y0, srcp, dstp)
    yh = _mid(s[0], s[1], y0, dinv, W1, b1.reshape(1, D_H), W2)
    t = _agg(yh, srcp, dstp)
    return _fin(t[0], t[1], yh, dinv, b2.reshape(1, D))


# speedup vs baseline: 1.0656x; 1.0656x over previous
import jax
import jax.numpy as jnp
from jax import lax
from jax.experimental import pallas as pl
from jax.experimental.pallas import tpu as pltpu
from jax.experimental.pallas import tpu_sc as plsc

N = 10000
E = 320000
D = 128
D_H = 256

NC = 2
NS = 16
NW = NC * NS
L = 16

CHUNK = 128
NCH = 80
EPT = NCH * CHUNK
E_PAD = NW * EPT
NP = 10240
RPT = NP // NS
NSLAB = 2
HCH = NCH // NSLAB


def _deg_body(dstH, cntH, dst_v, cnt_v):
    cid = lax.axis_index("c")
    sid = lax.axis_index("s")
    w = cid * NS + sid
    pltpu.sync_copy(dstH.at[w], dst_v)

    zeros16 = jnp.zeros((L,), jnp.float32)

    @pl.loop(0, NP // L)
    def _zero(i):
        cnt_v[pl.ds(i * L, L)] = zeros16

    ones16 = jnp.ones((L,), jnp.float32)

    for p in range(NSLAB):
        @pl.loop(0, NCH // NSLAB)
        def _count(j):
            for k in range(CHUNK // L):
                idx = dst_v[p, j, pl.ds(k * L, L)]
                plsc.addupdate_scatter(cnt_v, [idx], ones16)

    pltpu.sync_copy(cnt_v, cntH.at[w])


_deg = pl.kernel(
    _deg_body,
    out_type=jax.ShapeDtypeStruct((NW, NP), jnp.float32),
    mesh=plsc.VectorSubcoreMesh(core_axis_name="c", subcore_axis_name="s"),
    scratch_types=[
        pltpu.VMEM((NSLAB, HCH, CHUNK), jnp.int32),
        pltpu.VMEM((NP,), jnp.float32),
    ],
    compiler_params=pltpu.CompilerParams(needs_layout_passes=False),
)


def _agg_body(yH, srcH, dstH, outH, src_v, dst_v, rows0, rows1, S_sh,
              semg0, semg1, sems0, sems1):
    cid = lax.axis_index("c")
    sid = lax.axis_index("s")
    w = cid * NS + sid

    for p in range(NSLAB):
        pltpu.sync_copy(srcH.at[w].at[p], src_v)
        pltpu.sync_copy(dstH.at[w].at[p], dst_v)
        pltpu.async_copy(yH.at[src_v.at[0]], rows0, semg0)

        if p == 0:
            zeros16 = jnp.zeros((L,), jnp.float32)

            @pl.loop(0, CHUNK)
            def _zero(r):
                for k in range(D // L):
                    rows1[r, pl.ds(k * L, L)] = zeros16

            for t in range(RPT // CHUNK):
                pltpu.async_copy(
                    rows1, S_sh.at[pl.ds(sid * RPT + t * CHUNK, CHUNK)],
                    sems1)
            for t in range(RPT // CHUNK):
                pltpu.make_async_copy(
                    rows1, S_sh.at[pl.ds(sid * RPT + t * CHUNK, CHUNK)],
                    sems1).wait()
            plsc.subcore_barrier()

        @pl.loop(0, HCH, step=2)
        def _edges(j):
            pltpu.make_async_copy(yH.at[src_v.at[j]], rows0, semg0).wait()
            pltpu.async_copy(yH.at[src_v.at[j + 1]], rows1, semg1)
            pltpu.sync_copy(rows0, S_sh.at[dst_v.at[j]], add=True)
            pltpu.make_async_copy(yH.at[src_v.at[j + 1]], rows1, semg1).wait()

            @pl.when(j + 2 < HCH)
            def _():
                pltpu.async_copy(yH.at[src_v.at[j + 2]], rows0, semg0)

            pltpu.sync_copy(rows1, S_sh.at[dst_v.at[j + 1]], add=True)

    plsc.subcore_barrier()
    pltpu.sync_copy(S_sh.at[pl.ds(sid * RPT, RPT)],
                    outH.at[cid].at[pl.ds(sid * RPT, RPT)])


_agg = pl.kernel(
    _agg_body,
    out_type=jax.ShapeDtypeStruct((NC, NP, D), jnp.float32),
    mesh=plsc.VectorSubcoreMesh(core_axis_name="c", subcore_axis_name="s"),
    scratch_types=[
        pltpu.VMEM((HCH, CHUNK), jnp.int32),
        pltpu.VMEM((HCH, CHUNK), jnp.int32),
        pltpu.VMEM((CHUNK, D), jnp.float32),
        pltpu.VMEM((CHUNK, D), jnp.float32),
        pltpu.VMEM_SHARED((NP, D), jnp.float32),
        pltpu.SemaphoreType.DMA,
        pltpu.SemaphoreType.DMA,
        pltpu.SemaphoreType.DMA,
        pltpu.SemaphoreType.DMA,
    ],
)


PBLK = 512


def _prep_body(cnt_ref, x_ref, y_ref, dinv_ref):
    deg = jnp.sum(cnt_ref[...], axis=0) + 1.0
    dinv = lax.rsqrt(deg)[:, None]
    dinv_ref[...] = dinv
    y_ref[...] = x_ref[...] * dinv


_prep = pl.pallas_call(
    _prep_body,
    grid=(NP // PBLK,),
    in_specs=[
        pl.BlockSpec((NW, PBLK), lambda i: (0, i)),
        pl.BlockSpec((PBLK, D), lambda i: (i, 0)),
    ],
    out_specs=[
        pl.BlockSpec((PBLK, D), lambda i: (i, 0)),
        pl.BlockSpec((PBLK, 1), lambda i: (i, 0)),
    ],
    out_shape=[
        jax.ShapeDtypeStruct((NP, D), jnp.float32),
        jax.ShapeDtypeStruct((NP, 1), jnp.float32),
    ],
)


def _elu(v):
    return jnp.where(v > 0, v, jnp.exp(jnp.minimum(v, 0.0)) - 1.0)


MBLK = 512


def _mid_body(s0, s1, y0, dinv, w1, b1, w2, yh):
    a = (s0[...] + s1[...] + y0[...]) * dinv[...]
    h = _elu(jnp.dot(a, w1[...], preferred_element_type=jnp.float32)
             + b1[...])
    yh[...] = jnp.dot(h, w2[...],
                      preferred_element_type=jnp.float32) * dinv[...]


_mid = pl.pallas_call(
    _mid_body,
    grid=(NP // MBLK,),
    in_specs=[
        pl.BlockSpec((MBLK, D), lambda i: (i, 0)),
        pl.BlockSpec((MBLK, D), lambda i: (i, 0)),
        pl.BlockSpec((MBLK, D), lambda i: (i, 0)),
        pl.BlockSpec((MBLK, 1), lambda i: (i, 0)),
        pl.BlockSpec((D, D_H), lambda i: (0, 0)),
        pl.BlockSpec((1, D_H), lambda i: (0, 0)),
        pl.BlockSpec((D_H, D), lambda i: (0, 0)),
    ],
    out_specs=pl.BlockSpec((MBLK, D), lambda i: (i, 0)),
    out_shape=jax.ShapeDtypeStruct((NP, D), jnp.float32),
)


FBLK = 1000


def _fin_body(t0, t1, yh, dinv, b2, out):
    v = (t0[...] + t1[...] + yh[...]) * dinv[...] + b2[...]
    out[...] = _elu(v)


_fin = pl.pallas_call(
    _fin_body,
    grid=(N // FBLK,),
    in_specs=[
        pl.BlockSpec((FBLK, D), lambda i: (i, 0)),
        pl.BlockSpec((FBLK, D), lambda i: (i, 0)),
        pl.BlockSpec((FBLK, D), lambda i: (i, 0)),
        pl.BlockSpec((FBLK, 1), lambda i: (i, 0)),
        pl.BlockSpec((1, D), lambda i: (0, 0)),
    ],
    out_specs=pl.BlockSpec((FBLK, D), lambda i: (i, 0)),
    out_shape=jax.ShapeDtypeStruct((N, D), jnp.float32),
)


def kernel(x, edge_index, W1, b1, W2, b2):
    src = edge_index[0]
    dst = edge_index[1]
    pad = E_PAD - E
    pad_idx = (N + (jnp.arange(pad, dtype=jnp.int32) % (NP - N)))
    srcp = jnp.concatenate([src, pad_idx]).reshape(NW, NSLAB, HCH, CHUNK)
    dstp = jnp.concatenate([dst, pad_idx]).reshape(NW, NSLAB, HCH, CHUNK)
    x_pad = jnp.pad(x, ((0, NP - N), (0, 0)))

    cnt = _deg(dstp)
    y0, dinv = _prep(cnt, x_pad)
    s = _agg(y0, srcp, dstp)
    yh = _mid(s[0], s[1], y0, dinv, W1, b1.reshape(1, D_H), W2)
    t = _agg(yh, srcp, dstp)
    return _fin(t[0], t[1], yh, dinv, b2.reshape(1, D))


# speedup vs baseline: 1.2101x; 1.1356x over previous
import jax
import jax.numpy as jnp
from jax import lax
from jax.experimental import pallas as pl
from jax.experimental.pallas import tpu as pltpu
from jax.experimental.pallas import tpu_sc as plsc

N = 10000
E = 320000
D = 128
D_H = 256

NC = 2
NS = 16
NW = NC * NS
L = 16

CHUNK = 128
NCH = 80
EPT = NCH * CHUNK
E_PAD = NW * EPT
NP = 10240
RPT = NP // NS
NSLAB = 2
HCH = NCH // NSLAB


def _deg_body(dstH, cntH, dst_v, cnt_v):
    cid = lax.axis_index("c")
    sid = lax.axis_index("s")
    w = cid * NS + sid
    pltpu.sync_copy(dstH.at[w], dst_v)

    zeros16 = jnp.zeros((L,), jnp.float32)

    @pl.loop(0, NP // L)
    def _zero(i):
        cnt_v[pl.ds(i * L, L)] = zeros16

    ones16 = jnp.ones((L,), jnp.float32)

    for p in range(NSLAB):
        @pl.loop(0, NCH // NSLAB)
        def _count(j):
            for k in range(CHUNK // L):
                idx = dst_v[p, j, pl.ds(k * L, L)]
                plsc.addupdate_scatter(cnt_v, [idx], ones16)

    pltpu.sync_copy(cnt_v, cntH.at[w])


_deg = pl.kernel(
    _deg_body,
    out_type=jax.ShapeDtypeStruct((NW, NP), jnp.float32),
    mesh=plsc.VectorSubcoreMesh(core_axis_name="c", subcore_axis_name="s"),
    scratch_types=[
        pltpu.VMEM((NSLAB, HCH, CHUNK), jnp.int32),
        pltpu.VMEM((NP,), jnp.float32),
    ],
    compiler_params=pltpu.CompilerParams(needs_layout_passes=False),
)


def _agg_body(yH, srcH, dstH, outH, src_v, dst_v, rows0, rows1, S_sh,
              semg0, semg1, sems0, sems1):
    cid = lax.axis_index("c")
    sid = lax.axis_index("s")
    w = cid * NS + sid

    for p in range(NSLAB):
        pltpu.sync_copy(srcH.at[w].at[p], src_v)
        pltpu.sync_copy(dstH.at[w].at[p], dst_v)
        pltpu.async_copy(yH.at[src_v.at[0]], rows0, semg0)

        if p == 0:
            zeros16 = jnp.zeros((L,), jnp.float32)

            @pl.loop(0, CHUNK)
            def _zero(r):
                for k in range(D // L):
                    rows1[r, pl.ds(k * L, L)] = zeros16

            for t in range(RPT // CHUNK):
                pltpu.async_copy(
                    rows1, S_sh.at[pl.ds(sid * RPT + t * CHUNK, CHUNK)],
                    sems1)
            for t in range(RPT // CHUNK):
                pltpu.make_async_copy(
                    rows1, S_sh.at[pl.ds(sid * RPT + t * CHUNK, CHUNK)],
                    sems1).wait()
            plsc.subcore_barrier()

        pltpu.async_copy(yH.at[src_v.at[1]], rows1, semg1)

        @pl.loop(0, HCH, step=2)
        def _edges(j):
            pltpu.make_async_copy(yH.at[src_v.at[j]], rows0, semg0).wait()
            pltpu.sync_copy(rows0, S_sh.at[dst_v.at[j]], add=True)

            @pl.when(j + 2 < HCH)
            def _():
                pltpu.async_copy(yH.at[src_v.at[j + 2]], rows0, semg0)

            pltpu.make_async_copy(yH.at[src_v.at[j + 1]], rows1, semg1).wait()
            pltpu.sync_copy(rows1, S_sh.at[dst_v.at[j + 1]], add=True)

            @pl.when(j + 3 < HCH)
            def _():
                pltpu.async_copy(yH.at[src_v.at[j + 3]], rows1, semg1)

    plsc.subcore_barrier()
    pltpu.sync_copy(S_sh.at[pl.ds(sid * RPT, RPT)],
                    outH.at[cid].at[pl.ds(sid * RPT, RPT)])


_agg = pl.kernel(
    _agg_body,
    out_type=jax.ShapeDtypeStruct((NC, NP, D), jnp.float32),
    mesh=plsc.VectorSubcoreMesh(core_axis_name="c", subcore_axis_name="s"),
    scratch_types=[
        pltpu.VMEM((HCH, CHUNK), jnp.int32),
        pltpu.VMEM((HCH, CHUNK), jnp.int32),
        pltpu.VMEM((CHUNK, D), jnp.float32),
        pltpu.VMEM((CHUNK, D), jnp.float32),
        pltpu.VMEM_SHARED((NP, D), jnp.float32),
        pltpu.SemaphoreType.DMA,
        pltpu.SemaphoreType.DMA,
        pltpu.SemaphoreType.DMA,
        pltpu.SemaphoreType.DMA,
    ],
)


PBLK = 512


def _prep_body(cnt_ref, x_ref, y_ref, dinv_ref):
    deg = jnp.sum(cnt_ref[...], axis=0) + 1.0
    dinv = lax.rsqrt(deg)[:, None]
    dinv_ref[...] = dinv
    y_ref[...] = x_ref[...] * dinv


_prep = pl.pallas_call(
    _prep_body,
    grid=(NP // PBLK,),
    in_specs=[
        pl.BlockSpec((NW, PBLK), lambda i: (0, i)),
        pl.BlockSpec((PBLK, D), lambda i: (i, 0)),
    ],
    out_specs=[
        pl.BlockSpec((PBLK, D), lambda i: (i, 0)),
        pl.BlockSpec((PBLK, 1), lambda i: (i, 0)),
    ],
    out_shape=[
        jax.ShapeDtypeStruct((NP, D), jnp.float32),
        jax.ShapeDtypeStruct((NP, 1), jnp.float32),
    ],
)


def _elu(v):
    return jnp.where(v > 0, v, jnp.exp(jnp.minimum(v, 0.0)) - 1.0)


MBLK = 512


def _mid_body(s0, s1, y0, dinv, w1, b1, w2, yh):
    a = (s0[...] + s1[...] + y0[...]) * dinv[...]
    h = _elu(jnp.dot(a, w1[...], preferred_element_type=jnp.float32)
             + b1[...])
    yh[...] = jnp.dot(h, w2[...],
                      preferred_element_type=jnp.float32) * dinv[...]


_mid = pl.pallas_call(
    _mid_body,
    grid=(NP // MBLK,),
    in_specs=[
        pl.BlockSpec((MBLK, D), lambda i: (i, 0)),
        pl.BlockSpec((MBLK, D), lambda i: (i, 0)),
        pl.BlockSpec((MBLK, D), lambda i: (i, 0)),
        pl.BlockSpec((MBLK, 1), lambda i: (i, 0)),
        pl.BlockSpec((D, D_H), lambda i: (0, 0)),
        pl.BlockSpec((1, D_H), lambda i: (0, 0)),
        pl.BlockSpec((D_H, D), lambda i: (0, 0)),
    ],
    out_specs=pl.BlockSpec((MBLK, D), lambda i: (i, 0)),
    out_shape=jax.ShapeDtypeStruct((NP, D), jnp.float32),
)


FBLK = 1000


def _fin_body(t0, t1, yh, dinv, b2, out):
    v = (t0[...] + t1[...] + yh[...]) * dinv[...] + b2[...]
    out[...] = _elu(v)


_fin = pl.pallas_call(
    _fin_body,
    grid=(N // FBLK,),
    in_specs=[
        pl.BlockSpec((FBLK, D), lambda i: (i, 0)),
        pl.BlockSpec((FBLK, D), lambda i: (i, 0)),
        pl.BlockSpec((FBLK, D), lambda i: (i, 0)),
        pl.BlockSpec((FBLK, 1), lambda i: (i, 0)),
        pl.BlockSpec((1, D), lambda i: (0, 0)),
    ],
    out_specs=pl.BlockSpec((FBLK, D), lambda i: (i, 0)),
    out_shape=jax.ShapeDtypeStruct((N, D), jnp.float32),
)


def kernel(x, edge_index, W1, b1, W2, b2):
    src = edge_index[0]
    dst = edge_index[1]
    pad = E_PAD - E
    pad_idx = (N + (jnp.arange(pad, dtype=jnp.int32) % (NP - N)))
    srcp = jnp.concatenate([src, pad_idx]).reshape(NW, NSLAB, HCH, CHUNK)
    dstp = jnp.concatenate([dst, pad_idx]).reshape(NW, NSLAB, HCH, CHUNK)
    x_pad = jnp.pad(x, ((0, NP - N), (0, 0)))

    cnt = _deg(dstp)
    y0, dinv = _prep(cnt, x_pad)
    s = _agg(y0, srcp, dstp)
    yh = _mid(s[0], s[1], y0, dinv, W1, b1.reshape(1, D_H), W2)
    t = _agg(yh, srcp, dstp)
    return _fin(t[0], t[1], yh, dinv, b2.reshape(1, D))


# speedup vs baseline: 1.2596x; 1.0409x over previous
import jax
import jax.numpy as jnp
from jax import lax
from jax.experimental import pallas as pl
from jax.experimental.pallas import tpu as pltpu
from jax.experimental.pallas import tpu_sc as plsc

N = 10000
E = 320000
D = 128
D_H = 256

NC = 2
NS = 16
NW = NC * NS
L = 16

CHUNK = 80
NCH = 128
EPT = NCH * CHUNK
E_PAD = NW * EPT
NP = 10240
RPT = NP // NS
NSLAB = 4
HCH = NCH // NSLAB
NBUF = 4


def _deg_body(dstH, cntH, dst_v, cnt_v):
    cid = lax.axis_index("c")
    sid = lax.axis_index("s")
    w = cid * NS + sid
    pltpu.sync_copy(dstH.at[w], dst_v)

    zeros16 = jnp.zeros((L,), jnp.float32)

    @pl.loop(0, NP // L)
    def _zero(i):
        cnt_v[pl.ds(i * L, L)] = zeros16

    ones16 = jnp.ones((L,), jnp.float32)

    for p in range(NSLAB):
        @pl.loop(0, NCH // NSLAB)
        def _count(j):
            for k in range(CHUNK // L):
                idx = dst_v[p, j, pl.ds(k * L, L)]
                plsc.addupdate_scatter(cnt_v, [idx], ones16)

    pltpu.sync_copy(cnt_v, cntH.at[w])


_deg = pl.kernel(
    _deg_body,
    out_type=jax.ShapeDtypeStruct((NW, NP), jnp.float32),
    mesh=plsc.VectorSubcoreMesh(core_axis_name="c", subcore_axis_name="s"),
    scratch_types=[
        pltpu.VMEM((NSLAB, HCH, CHUNK), jnp.int32),
        pltpu.VMEM((NP,), jnp.float32),
    ],
    compiler_params=pltpu.CompilerParams(needs_layout_passes=False),
)


def _agg_body(yH, srcH, dstH, outH, src_v, dst_v, r0, r1, r2, r3, S_sh,
              sg0, sg1, sg2, sg3, semz):
    rows = (r0, r1, r2, r3)
    semg = (sg0, sg1, sg2, sg3)
    cid = lax.axis_index("c")
    sid = lax.axis_index("s")
    w = cid * NS + sid

    for p in range(NSLAB):
        pltpu.sync_copy(srcH.at[w].at[p], src_v)
        pltpu.sync_copy(dstH.at[w].at[p], dst_v)
        for b in range(NBUF - 1):
            pltpu.async_copy(yH.at[src_v.at[b]], rows[b], semg[b])

        if p == 0:
            zeros16 = jnp.zeros((L,), jnp.float32)

            @pl.loop(0, CHUNK)
            def _zero(r):
                for k in range(D // L):
                    rows[NBUF - 1][r, pl.ds(k * L, L)] = zeros16

            for t in range(RPT // CHUNK):
                pltpu.async_copy(
                    rows[NBUF - 1],
                    S_sh.at[pl.ds(sid * RPT + t * CHUNK, CHUNK)], semz)
            for t in range(RPT // CHUNK):
                pltpu.make_async_copy(
                    rows[NBUF - 1],
                    S_sh.at[pl.ds(sid * RPT + t * CHUNK, CHUNK)], semz).wait()
            plsc.subcore_barrier()

        pltpu.async_copy(yH.at[src_v.at[NBUF - 1]], rows[NBUF - 1],
                         semg[NBUF - 1])

        @pl.loop(0, HCH, step=NBUF)
        def _edges(j):
            for b in range(NBUF):
                pltpu.make_async_copy(yH.at[src_v.at[j + b]], rows[b],
                                      semg[b]).wait()
                pltpu.sync_copy(rows[b], S_sh.at[dst_v.at[j + b]], add=True)

                @pl.when(j + b + NBUF < HCH)
                def _():
                    pltpu.async_copy(yH.at[src_v.at[j + b + NBUF]], rows[b],
                                     semg[b])

    plsc.subcore_barrier()
    pltpu.sync_copy(S_sh.at[pl.ds(sid * RPT, RPT)],
                    outH.at[cid].at[pl.ds(sid * RPT, RPT)])


_agg = pl.kernel(
    _agg_body,
    out_type=jax.ShapeDtypeStruct((NC, NP, D), jnp.float32),
    mesh=plsc.VectorSubcoreMesh(core_axis_name="c", subcore_axis_name="s"),
    scratch_types=[
        pltpu.VMEM((HCH, CHUNK), jnp.int32),
        pltpu.VMEM((HCH, CHUNK), jnp.int32),
        pltpu.VMEM((CHUNK, D), jnp.float32),
        pltpu.VMEM((CHUNK, D), jnp.float32),
        pltpu.VMEM((CHUNK, D), jnp.float32),
        pltpu.VMEM((CHUNK, D), jnp.float32),
        pltpu.VMEM_SHARED((NP, D), jnp.float32),
        pltpu.SemaphoreType.DMA,
        pltpu.SemaphoreType.DMA,
        pltpu.SemaphoreType.DMA,
        pltpu.SemaphoreType.DMA,
        pltpu.SemaphoreType.DMA,
    ],
)


PBLK = 512


def _prep_body(cnt_ref, x_ref, y_ref, dinv_ref):
    deg = jnp.sum(cnt_ref[...], axis=0) + 1.0
    dinv = lax.rsqrt(deg)[:, None]
    dinv_ref[...] = dinv
    y_ref[...] = x_ref[...] * dinv


_prep = pl.pallas_call(
    _prep_body,
    grid=(NP // PBLK,),
    in_specs=[
        pl.BlockSpec((NW, PBLK), lambda i: (0, i)),
        pl.BlockSpec((PBLK, D), lambda i: (i, 0)),
    ],
    out_specs=[
        pl.BlockSpec((PBLK, D), lambda i: (i, 0)),
        pl.BlockSpec((PBLK, 1), lambda i: (i, 0)),
    ],
    out_shape=[
        jax.ShapeDtypeStruct((NP, D), jnp.float32),
        jax.ShapeDtypeStruct((NP, 1), jnp.float32),
    ],
)


def _elu(v):
    return jnp.where(v > 0, v, jnp.exp(jnp.minimum(v, 0.0)) - 1.0)


MBLK = 512


def _mid_body(s0, s1, y0, dinv, w1, b1, w2, yh):
    a = (s0[...] + s1[...] + y0[...]) * dinv[...]
    h = _elu(jnp.dot(a, w1[...], preferred_element_type=jnp.float32)
             + b1[...])
    yh[...] = jnp.dot(h, w2[...],
                      preferred_element_type=jnp.float32) * dinv[...]


_mid = pl.pallas_call(
    _mid_body,
    grid=(NP // MBLK,),
    in_specs=[
        pl.BlockSpec((MBLK, D), lambda i: (i, 0)),
        pl.BlockSpec((MBLK, D), lambda i: (i, 0)),
        pl.BlockSpec((MBLK, D), lambda i: (i, 0)),
        pl.BlockSpec((MBLK, 1), lambda i: (i, 0)),
        pl.BlockSpec((D, D_H), lambda i: (0, 0)),
        pl.BlockSpec((1, D_H), lambda i: (0, 0)),
        pl.BlockSpec((D_H, D), lambda i: (0, 0)),
    ],
    out_specs=pl.BlockSpec((MBLK, D), lambda i: (i, 0)),
    out_shape=jax.ShapeDtypeStruct((NP, D), jnp.float32),
)


FBLK = 1000


def _fin_body(t0, t1, yh, dinv, b2, out):
    v = (t0[...] + t1[...] + yh[...]) * dinv[...] + b2[...]
    out[...] = _elu(v)


_fin = pl.pallas_call(
    _fin_body,
    grid=(N // FBLK,),
    in_specs=[
        pl.BlockSpec((FBLK, D), lambda i: (i, 0)),
        pl.BlockSpec((FBLK, D), lambda i: (i, 0)),
        pl.BlockSpec((FBLK, D), lambda i: (i, 0)),
        pl.BlockSpec((FBLK, 1), lambda i: (i, 0)),
        pl.BlockSpec((1, D), lambda i: (0, 0)),
    ],
    out_specs=pl.BlockSpec((FBLK, D), lambda i: (i, 0)),
    out_shape=jax.ShapeDtypeStruct((N, D), jnp.float32),
)


def kernel(x, edge_index, W1, b1, W2, b2):
    src = edge_index[0]
    dst = edge_index[1]
    pad = E_PAD - E
    pad_idx = (N + (jnp.arange(pad, dtype=jnp.int32) % (NP - N)))
    srcp = jnp.concatenate([src, pad_idx]).reshape(NW, NSLAB, HCH, CHUNK)
    dstp = jnp.concatenate([dst, pad_idx]).reshape(NW, NSLAB, HCH, CHUNK)
    x_pad = jnp.pad(x, ((0, NP - N), (0, 0)))

    cnt = _deg(dstp)
    y0, dinv = _prep(cnt, x_pad)
    s = _agg(y0, srcp, dstp)
    yh = _mid(s[0], s[1], y0, dinv, W1, b1.reshape(1, D_H), W2)
    t = _agg(yh, srcp, dstp)
    return _fin(t[0], t[1], yh, dinv, b2.reshape(1, D))


# speedup vs baseline: 1.3796x; 1.0953x over previous
import jax
import jax.numpy as jnp
from jax import lax
from jax.experimental import pallas as pl
from jax.experimental.pallas import tpu as pltpu
from jax.experimental.pallas import tpu_sc as plsc

N = 10000
E = 320000
D = 128
D_H = 256

NC = 2
NS = 16
NW = NC * NS
L = 16

CHUNK = 80
NCH = 128
EPT = NCH * CHUNK
E_PAD = NW * EPT
NP = 10240
RPT = NP // NS
NSLAB = 4
HCH = NCH // NSLAB
NBUF = 4


def _deg_body(dstH, cntH, dst_v, cnt_v):
    cid = lax.axis_index("c")
    sid = lax.axis_index("s")
    w = cid * NS + sid
    pltpu.sync_copy(dstH.at[w], dst_v)

    zeros16 = jnp.zeros((L,), jnp.float32)

    @pl.loop(0, NP // L)
    def _zero(i):
        cnt_v[pl.ds(i * L, L)] = zeros16

    ones16 = jnp.ones((L,), jnp.float32)

    for p in range(NSLAB):
        @pl.loop(0, NCH // NSLAB)
        def _count(j):
            for k in range(CHUNK // L):
                idx = dst_v[p, j, pl.ds(k * L, L)]
                plsc.addupdate_scatter(cnt_v, [idx], ones16)

    pltpu.sync_copy(cnt_v, cntH.at[w])


_deg = pl.kernel(
    _deg_body,
    out_type=jax.ShapeDtypeStruct((NW, NP), jnp.float32),
    mesh=plsc.VectorSubcoreMesh(core_axis_name="c", subcore_axis_name="s"),
    scratch_types=[
        pltpu.VMEM((NSLAB, HCH, CHUNK), jnp.int32),
        pltpu.VMEM((NP,), jnp.float32),
    ],
    compiler_params=pltpu.CompilerParams(needs_layout_passes=False),
)


def _agg_body(yH, srcH, dstH, outH, src_v, dst_v, r0, r1, r2, r3, S_sh,
              sg0, sg1, sg2, sg3, semz):
    rows = (r0, r1, r2, r3)
    semg = (sg0, sg1, sg2, sg3)
    cid = lax.axis_index("c")
    sid = lax.axis_index("s")
    w = cid * NS + sid

    for p in range(NSLAB):
        pltpu.sync_copy(srcH.at[w].at[p], src_v)
        pltpu.sync_copy(dstH.at[w].at[p], dst_v)
        for b in range(NBUF - 1):
            pltpu.async_copy(yH.at[src_v.at[b]], rows[b], semg[b])

        if p == 0:
            zeros16 = jnp.zeros((L,), jnp.float32)

            @pl.loop(0, CHUNK)
            def _zero(r):
                for k in range(D // L):
                    rows[NBUF - 1][r, pl.ds(k * L, L)] = zeros16

            for t in range(RPT // CHUNK):
                pltpu.async_copy(
                    rows[NBUF - 1],
                    S_sh.at[pl.ds(sid * RPT + t * CHUNK, CHUNK)], semz)
            for t in range(RPT // CHUNK):
                pltpu.make_async_copy(
                    rows[NBUF - 1],
                    S_sh.at[pl.ds(sid * RPT + t * CHUNK, CHUNK)], semz).wait()
            plsc.subcore_barrier()

        pltpu.async_copy(yH.at[src_v.at[NBUF - 1]], rows[NBUF - 1],
                         semg[NBUF - 1])

        @pl.loop(0, HCH, step=NBUF)
        def _edges(j):
            for b in range(NBUF):
                pltpu.make_async_copy(yH.at[src_v.at[j + b]], rows[b],
                                      semg[b]).wait()
                pltpu.sync_copy(rows[b], S_sh.at[dst_v.at[j + b]], add=True)

                @pl.when(j + b + NBUF < HCH)
                def _():
                    pltpu.async_copy(yH.at[src_v.at[j + b + NBUF]], rows[b],
                                     semg[b])

    plsc.subcore_barrier()
    pltpu.sync_copy(S_sh.at[pl.ds(sid * RPT, RPT)],
                    outH.at[cid].at[pl.ds(sid * RPT, RPT)])


_agg = pl.kernel(
    _agg_body,
    out_type=jax.ShapeDtypeStruct((NC, NP, D), jnp.float32),
    mesh=plsc.VectorSubcoreMesh(core_axis_name="c", subcore_axis_name="s"),
    scratch_types=[
        pltpu.VMEM((HCH, CHUNK), jnp.int32),
        pltpu.VMEM((HCH, CHUNK), jnp.int32),
        pltpu.VMEM((CHUNK, D), jnp.float32),
        pltpu.VMEM((CHUNK, D), jnp.float32),
        pltpu.VMEM((CHUNK, D), jnp.float32),
        pltpu.VMEM((CHUNK, D), jnp.float32),
        pltpu.VMEM_SHARED((NP, D), jnp.float32),
        pltpu.SemaphoreType.DMA,
        pltpu.SemaphoreType.DMA,
        pltpu.SemaphoreType.DMA,
        pltpu.SemaphoreType.DMA,
        pltpu.SemaphoreType.DMA,
    ],
)


PBLK = 1024


def _prep_body(cnt_ref, x_ref, y_ref, dinv_ref):
    deg = jnp.sum(cnt_ref[...], axis=0) + 1.0
    dinv = lax.rsqrt(deg)[:, None]
    dinv_ref[...] = dinv
    y_ref[...] = x_ref[...] * dinv


_prep = pl.pallas_call(
    _prep_body,
    grid=(NP // PBLK,),
    in_specs=[
        pl.BlockSpec((NW, PBLK), lambda i: (0, i)),
        pl.BlockSpec((PBLK, D), lambda i: (i, 0)),
    ],
    out_specs=[
        pl.BlockSpec((PBLK, D), lambda i: (i, 0)),
        pl.BlockSpec((PBLK, 1), lambda i: (i, 0)),
    ],
    out_shape=[
        jax.ShapeDtypeStruct((NP, D), jnp.float32),
        jax.ShapeDtypeStruct((NP, 1), jnp.float32),
    ],
)


def _elu(v):
    return jnp.where(v > 0, v, jnp.exp(jnp.minimum(v, 0.0)) - 1.0)


MBLK = 1000


def _mid_body(s, y0, dinv, w1, b1, w2, yh):
    a = (s[0] + s[1] + y0[...]) * dinv[...]
    h = _elu(jnp.dot(a, w1[...], preferred_element_type=jnp.float32)
             + b1[...])
    yh[...] = jnp.dot(h, w2[...],
                      preferred_element_type=jnp.float32) * dinv[...]


_mid = pl.pallas_call(
    _mid_body,
    grid=(N // MBLK,),
    in_specs=[
        pl.BlockSpec((NC, MBLK, D), lambda i: (0, i, 0)),
        pl.BlockSpec((MBLK, D), lambda i: (i, 0)),
        pl.BlockSpec((MBLK, 1), lambda i: (i, 0)),
        pl.BlockSpec((D, D_H), lambda i: (0, 0)),
        pl.BlockSpec((1, D_H), lambda i: (0, 0)),
        pl.BlockSpec((D_H, D), lambda i: (0, 0)),
    ],
    out_specs=pl.BlockSpec((MBLK, D), lambda i: (i, 0)),
    out_shape=jax.ShapeDtypeStruct((NP, D), jnp.float32),
)


FBLK = 2000


def _fin_body(t, yh, dinv, b2, out):
    v = (t[0] + t[1] + yh[...]) * dinv[...] + b2[...]
    out[...] = _elu(v)


_fin = pl.pallas_call(
    _fin_body,
    grid=(N // FBLK,),
    in_specs=[
        pl.BlockSpec((NC, FBLK, D), lambda i: (0, i, 0)),
        pl.BlockSpec((FBLK, D), lambda i: (i, 0)),
        pl.BlockSpec((FBLK, 1), lambda i: (i, 0)),
        pl.BlockSpec((1, D), lambda i: (0, 0)),
    ],
    out_specs=pl.BlockSpec((FBLK, D), lambda i: (i, 0)),
    out_shape=jax.ShapeDtypeStruct((N, D), jnp.float32),
)


def kernel(x, edge_index, W1, b1, W2, b2):
    src = edge_index[0]
    dst = edge_index[1]
    pad = E_PAD - E
    pad_idx = (N + (jnp.arange(pad, dtype=jnp.int32) % (NP - N)))
    srcp = jnp.concatenate([src, pad_idx]).reshape(NW, NSLAB, HCH, CHUNK)
    dstp = jnp.concatenate([dst, pad_idx]).reshape(NW, NSLAB, HCH, CHUNK)

    cnt = _deg(dstp)
    y0, dinv = _prep(cnt, x)
    s = _agg(y0, srcp, dstp)
    yh = _mid(s, y0, dinv, W1, b1.reshape(1, D_H), W2)
    t = _agg(yh, srcp, dstp)
    return _fin(t, yh, dinv, b2.reshape(1, D))
